# baseline (speedup 1.0000x reference)
import functools

import jax
import jax.numpy as jnp
from jax import lax
from jax.experimental import pallas as pl
from jax.experimental.pallas import tpu as pltpu
from jax.experimental.pallas import tpu_sc as plsc

IDXW = 128
NBUF = 2
LANES = 16
DPAD = 128
SLAB = 16


def _build(B, L, V, D, NC, NS):
    NW = NC * NS
    seqs_w = B // NW
    n_chunks = seqs_w
    rem = L - IDXW

    mesh = plsc.VectorSubcoreMesh(
        core_axis_name="c", subcore_axis_name="s",
        num_cores=NC, num_subcores=NS)

    @functools.partial(
        pl.kernel,
        out_type=jax.ShapeDtypeStruct((B, L, D), jnp.float32),
        mesh=mesh,
        scratch_types=[
            pltpu.VMEM((2, SLAB, L), jnp.int32),
            pltpu.VMEM((L // 2, DPAD), jnp.float32),
            pltpu.VMEM((NBUF, L, DPAD), jnp.float32),
            pltpu.VMEM((NBUF, L, D), jnp.float32),
            pltpu.VMEM((NBUF, 2, IDXW), jnp.int32),
            pltpu.SemaphoreType.DMA,
            pltpu.SemaphoreType.DMA,
            pltpu.SemaphoreType.DMA,
            pltpu.SemaphoreType.DMA,
        ],
        compiler_params=pltpu.CompilerParams(use_tc_tiling_on_sc=True),
    )
    def body(idx_hbm, table_hbm, pos_hbm, out_hbm,
             idx_v, pos_v, rows_v, stage_v, gidx_v, g0, g1, o0, o1):
        gsems = (g0, g1)
        osems = (o0, o1)
        wid = lax.axis_index("s") * NC + lax.axis_index("c")
        seq_base = wid * seqs_w

        pltpu.sync_copy(pos_hbm, pos_v)

        def load_slab(j):
            pltpu.sync_copy(
                idx_hbm.at[pl.ds(seq_base + j * SLAB, SLAB)],
                idx_v.at[j & 1])

        def fire_gather(c, b):
            sb = (c // SLAB) & 1
            row = lax.rem(c, SLAB)
            for k in range(IDXW // LANES):
                gidx_v[b, 0, pl.ds(k * LANES, LANES)] = (
                    idx_v[sb, row, pl.ds(k * LANES, LANES)] >> 1)
            for k in range(4):
                gidx_v[b, 1, pl.ds(k * LANES, LANES)] = (
                    idx_v[sb, row, pl.ds(IDXW + k * LANES, LANES)] >> 1)
            gidx_v[b, 1, pl.ds(rem - LANES, LANES)] = (
                idx_v[sb, row, pl.ds(L - LANES, LANES)] >> 1)
            pltpu.async_copy(
                table_hbm.at[gidx_v.at[b, 0]],
                rows_v.at[b, pl.ds(0, IDXW)],
                gsems[b])
            pltpu.async_copy(
                table_hbm.at[gidx_v.at[b, 1, pl.ds(0, rem)]],
                rows_v.at[b, pl.ds(IDXW, rem)],
                gsems[b])

        def drain_gather(b):
            pltpu.make_async_copy(
                table_hbm.at[pl.ds(0, IDXW)],
                rows_v.at[b, pl.ds(0, IDXW)], gsems[b]).wait()
            pltpu.make_async_copy(
                table_hbm.at[pl.ds(0, rem)],
                rows_v.at[b, pl.ds(IDXW, rem)], gsems[b]).wait()

        def drain_store(b):
            pltpu.make_async_copy(
                stage_v.at[b], out_hbm.at[0], osems[b]).wait()

        def trip(c, b):
            @pl.when(jnp.logical_and(lax.rem(c + 2, SLAB) == 0,
                                     c + 2 < n_chunks))
            def _():
                load_slab((c + 2) // SLAB)

            drain_gather(b)

            @pl.when(c >= 2)
            def _():
                drain_store(b)

            sb = (c // SLAB) & 1
            row = lax.rem(c, SLAB)

            def add_rows(r0, js):
                pv = (idx_v[sb, row, pl.ds(r0, LANES)] & 1) * D
                for j in js:
                    r = r0 + j
                    for q in range(D // LANES):
                        stage_v[b, r, pl.ds(q * LANES, LANES)] = (
                            rows_v[b, r, pl.ds(pv[j] + q * LANES, LANES)]
                            + pos_v[(r0 >> 1) + (j >> 1),
                                    pl.ds((j & 1) * D + q * LANES, LANES)])

            @pl.loop(0, (L // LANES) * LANES, step=LANES)
            def _add(r0):
                add_rows(r0, range(LANES))

            add_rows(L - LANES, range(LANES - (L % LANES), LANES))

            @pl.when(c + 2 < n_chunks)
            def _():
                fire_gather(c + 2, b)

            pltpu.async_copy(
                stage_v.at[b], out_hbm.at[seq_base + c], osems[b])

        load_slab(0)
        fire_gather(0, 0)
        fire_gather(1, 1)

        @pl.loop(0, n_chunks, step=NBUF)
        def _outer(t):
            for db in range(NBUF):
                trip(t + db, db)

        drain_store(0)
        drain_store(1)

    return body


def kernel(inputs, token_table, pos_table):
    B, L = inputs.shape
    V, D = token_table.shape
    info = plsc.get_sparse_core_info()
    NC, NS = info.num_cores, info.num_subcores
    tblp = token_table.reshape(V // 2, DPAD)
    pos2 = pos_table.reshape(L // 2, DPAD)
    out = _build(B, L, V, D, NC, NS)(
        inputs.astype(jnp.int32), tblp, pos2)
    return out
